# baseline (speedup 1.0000x reference)
import functools

import jax
import jax.numpy as jnp
from jax import lax
from jax.experimental import pallas as pl
from jax.experimental.pallas import tpu as pltpu
from jax.experimental.pallas import tpu_sc as plsc


def kernel(x, emb_table):
    L = x.shape[1]
    D = emb_table.shape[1]

    info = plsc.get_sparse_core_info()
    num_workers = info.num_cores * info.num_subcores
    rows_per_worker = L // num_workers

    mesh = plsc.VectorSubcoreMesh(core_axis_name="c", subcore_axis_name="s")

    @functools.partial(
        pl.kernel,
        mesh=mesh,
        out_type=jax.ShapeDtypeStruct((L, D), jnp.float32),
    )
    def copy_rows(table_hbm, out_hbm):
        wid = lax.axis_index("s") * info.num_cores + lax.axis_index("c")
        base = wid * rows_per_worker
        pltpu.sync_copy(
            table_hbm.at[pl.ds(base, rows_per_worker)],
            out_hbm.at[pl.ds(base, rows_per_worker)],
        )

    return copy_rows(emb_table)[None].astype(jnp.float32)


# speedup vs baseline: 20.6078x; 20.6078x over previous
import functools

import jax
import jax.numpy as jnp
from jax import lax
from jax.experimental import pallas as pl
from jax.experimental.pallas import tpu as pltpu
from jax.experimental.pallas import tpu_sc as plsc

_CHUNK_ROWS = 64


def kernel(x, emb_table):
    L = x.shape[1]
    D = emb_table.shape[1]

    info = plsc.get_sparse_core_info()
    num_workers = info.num_cores * info.num_subcores
    rows_per_worker = L // num_workers
    n_chunks = rows_per_worker // _CHUNK_ROWS

    mesh = plsc.VectorSubcoreMesh(core_axis_name="c", subcore_axis_name="s")

    @functools.partial(
        pl.kernel,
        mesh=mesh,
        out_type=jax.ShapeDtypeStruct((L, D), jnp.float32),
        scratch_types=[
            pltpu.VMEM((_CHUNK_ROWS, D), jnp.float32),
            pltpu.VMEM((_CHUNK_ROWS, D), jnp.float32),
            pltpu.SemaphoreType.DMA,
            pltpu.SemaphoreType.DMA,
            pltpu.SemaphoreType.DMA,
            pltpu.SemaphoreType.DMA,
        ],
    )
    def copy_rows(table_hbm, out_hbm, buf0, buf1, si0, si1, so0, so1):
        wid = lax.axis_index("s") * info.num_cores + lax.axis_index("c")
        base = wid * rows_per_worker
        bufs = (buf0, buf1)
        sin = (si0, si1)
        sout = (so0, so1)

        in_dma = [None] * n_chunks
        out_dma = [None] * n_chunks
        for i in range(n_chunks):
            b = i % 2
            if i >= 2:
                out_dma[i - 2].wait()
            in_dma[i] = pltpu.async_copy(
                table_hbm.at[pl.ds(base + i * _CHUNK_ROWS, _CHUNK_ROWS)],
                bufs[b],
                sin[b],
            )
            in_dma[i].wait()
            out_dma[i] = pltpu.async_copy(
                bufs[b],
                out_hbm.at[pl.ds(base + i * _CHUNK_ROWS, _CHUNK_ROWS)],
                sout[b],
            )
        for i in range(max(n_chunks - 2, 0), n_chunks):
            out_dma[i].wait()

    return copy_rows(emb_table)[None].astype(jnp.float32)


# speedup vs baseline: 21.5528x; 1.0459x over previous
import functools

import jax
import jax.numpy as jnp
from jax import lax
from jax.experimental import pallas as pl
from jax.experimental.pallas import tpu as pltpu
from jax.experimental.pallas import tpu_sc as plsc

_CHUNK_ROWS = 32
_NBUF = 4


def kernel(x, emb_table):
    L = x.shape[1]
    D = emb_table.shape[1]

    info = plsc.get_sparse_core_info()
    num_workers = info.num_cores * info.num_subcores
    rows_per_worker = L // num_workers
    n_chunks = rows_per_worker // _CHUNK_ROWS

    mesh = plsc.VectorSubcoreMesh(core_axis_name="c", subcore_axis_name="s")

    @functools.partial(
        pl.kernel,
        mesh=mesh,
        out_type=jax.ShapeDtypeStruct((1, L, D), jnp.float32),
        scratch_types=(
            [pltpu.VMEM((_CHUNK_ROWS, D), jnp.float32) for _ in range(_NBUF)]
            + [pltpu.SemaphoreType.DMA for _ in range(2 * _NBUF)]
        ),
    )
    def copy_rows(table_hbm, out_hbm, *scratch):
        bufs = scratch[:_NBUF]
        sin = scratch[_NBUF : 2 * _NBUF]
        sout = scratch[2 * _NBUF :]
        wid = lax.axis_index("s") * info.num_cores + lax.axis_index("c")
        base = wid * rows_per_worker

        def read(i):
            return pltpu.async_copy(
                table_hbm.at[pl.ds(base + i * _CHUNK_ROWS, _CHUNK_ROWS)],
                bufs[i % _NBUF],
                sin[i % _NBUF],
            )

        def write(i):
            return pltpu.async_copy(
                bufs[i % _NBUF],
                out_hbm.at[0, pl.ds(base + i * _CHUNK_ROWS, _CHUNK_ROWS)],
                sout[i % _NBUF],
            )

        in_dma = [None] * n_chunks
        out_dma = [None] * n_chunks
        for i in range(min(_NBUF, n_chunks)):
            in_dma[i] = read(i)
        for i in range(n_chunks):
            in_dma[i].wait()
            out_dma[i] = write(i)
            j = i + _NBUF
            if j < n_chunks:
                out_dma[j - _NBUF].wait()
                in_dma[j] = read(j)
        for i in range(max(n_chunks - _NBUF, 0), n_chunks):
            out_dma[i].wait()

    return copy_rows(emb_table)


# speedup vs baseline: 21.8564x; 1.0141x over previous
import functools

import jax
import jax.numpy as jnp
from jax import lax
from jax.experimental import pallas as pl
from jax.experimental.pallas import tpu as pltpu
from jax.experimental.pallas import tpu_sc as plsc

_CHUNK_ROWS = 32
_NBUF = 5


def kernel(x, emb_table):
    L = x.shape[1]
    D = emb_table.shape[1]

    info = plsc.get_sparse_core_info()
    num_workers = info.num_cores * info.num_subcores
    rows_per_worker = L // num_workers
    n_chunks = rows_per_worker // _CHUNK_ROWS

    mesh = plsc.VectorSubcoreMesh(core_axis_name="c", subcore_axis_name="s")

    @functools.partial(
        pl.kernel,
        mesh=mesh,
        out_type=jax.ShapeDtypeStruct((1, L, D), jnp.float32),
        scratch_types=(
            [pltpu.VMEM((_CHUNK_ROWS, D), jnp.float32) for _ in range(_NBUF)]
            + [pltpu.SemaphoreType.DMA for _ in range(2 * _NBUF)]
        ),
    )
    def copy_rows(table_hbm, out_hbm, *scratch):
        bufs = scratch[:_NBUF]
        sin = scratch[_NBUF : 2 * _NBUF]
        sout = scratch[2 * _NBUF :]
        wid = lax.axis_index("s") * info.num_cores + lax.axis_index("c")
        base = wid * rows_per_worker

        def read(i):
            return pltpu.async_copy(
                table_hbm.at[pl.ds(base + i * _CHUNK_ROWS, _CHUNK_ROWS)],
                bufs[i % _NBUF],
                sin[i % _NBUF],
            )

        def write(i):
            return pltpu.async_copy(
                bufs[i % _NBUF],
                out_hbm.at[0, pl.ds(base + i * _CHUNK_ROWS, _CHUNK_ROWS)],
                sout[i % _NBUF],
            )

        in_dma = [None] * n_chunks
        out_dma = [None] * n_chunks
        for i in range(min(_NBUF, n_chunks)):
            in_dma[i] = read(i)
        for i in range(n_chunks):
            in_dma[i].wait()
            out_dma[i] = write(i)
            j = i + _NBUF
            if j < n_chunks:
                out_dma[j - _NBUF].wait()
                in_dma[j] = read(j)
        for i in range(max(n_chunks - _NBUF, 0), n_chunks):
            out_dma[i].wait()

    return copy_rows(emb_table)
